# Initial kernel scaffold; baseline (speedup 1.0000x reference)
import jax
import jax.numpy as jnp
from jax.experimental import pallas as pl


def kernel(x, edge_index, W0, as0, ad0, b0, g0, be0, W1, as1, ad1, b1, g1, be1, W2, as2, ad2, b2, g2, be2):
    raise NotImplementedError("write your pallas kernel here")


# speedup vs baseline: 58.2628x; 58.2628x over previous
import functools

import numpy as np
import jax
import jax.numpy as jnp
from jax import lax
from jax.experimental import pallas as pl
from jax.experimental.pallas import tpu as pltpu
from jax.experimental.pallas import tpu_sc as plsc

N = 10000
D = 128
L = 16
NC = 2
NS = 16
NW = NC * NS
K = 64
NACC = 10240
RB = 400


def _prep_body(h_ref, w_ref, sd_ref, hp_ref, asd_ref):
    hp = jnp.dot(h_ref[...], w_ref[...], preferred_element_type=jnp.float32)
    hp_ref[...] = hp
    asd_ref[...] = jnp.dot(hp, sd_ref[...], preferred_element_type=jnp.float32)


def _tc_prep(h, W, SD):
    return pl.pallas_call(
        _prep_body,
        grid=(N // RB,),
        in_specs=[
            pl.BlockSpec((RB, D), lambda i: (i, 0)),
            pl.BlockSpec((D, D), lambda i: (0, 0)),
            pl.BlockSpec((D, D), lambda i: (0, 0)),
        ],
        out_specs=[pl.BlockSpec((RB, D), lambda i: (i, 0))] * 2,
        out_shape=[jax.ShapeDtypeStruct((N, D), jnp.float32)] * 2,
    )(h, W, SD)


def _make_post_body(with_prep):
    def body(hprev_ref, n0_ref, n1_ref, d0_ref, d1_ref, rm_ref, b_ref,
             g_ref, be_ref, *rest):
        if with_prep:
            w_ref, sd_ref, hn_ref, hp_ref, asd_ref = rest
        else:
            (hn_ref,) = rest
        den = d0_ref[...] + d1_ref[...]
        den_exp = jnp.dot(den, rm_ref[...], preferred_element_type=jnp.float32)
        gat = (n0_ref[...] + n1_ref[...]) / (den_exp + 1e-16) + b_ref[...]
        mu = jnp.mean(gat, axis=-1, keepdims=True)
        xc = gat - mu
        var = jnp.mean(xc * xc, axis=-1, keepdims=True)
        y = xc * lax.rsqrt(var + 1e-5) * g_ref[...] + be_ref[...]
        z = hprev_ref[...] + y
        hn = jnp.where(z > 0, z, jnp.exp(jnp.minimum(z, 0.0)) - 1.0)
        hn_ref[...] = hn
        if with_prep:
            hp = jnp.dot(hn, w_ref[...], preferred_element_type=jnp.float32)
            hp_ref[...] = hp
            asd_ref[...] = jnp.dot(hp, sd_ref[...],
                                   preferred_element_type=jnp.float32)
    return body


def _tc_post(hprev, n0, n1, d0, d1, Rm, b, g, be, W=None, SD=None):
    with_prep = W is not None
    full = lambda i: (0, 0)
    row = lambda i: (i, 0)
    in_specs = [
        pl.BlockSpec((RB, D), row),
        pl.BlockSpec((RB, D), row),
        pl.BlockSpec((RB, D), row),
        pl.BlockSpec((RB, L), row),
        pl.BlockSpec((RB, L), row),
        pl.BlockSpec((L, D), full),
        pl.BlockSpec((1, D), full),
        pl.BlockSpec((1, D), full),
        pl.BlockSpec((1, D), full),
    ]
    args = [hprev, n0, n1, d0, d1, Rm, b.reshape(1, D), g.reshape(1, D),
            be.reshape(1, D)]
    if with_prep:
        in_specs += [pl.BlockSpec((D, D), full), pl.BlockSpec((D, D), full)]
        args += [W, SD]
        out_specs = [pl.BlockSpec((RB, D), row)] * 3
        out_shape = [jax.ShapeDtypeStruct((N, D), jnp.float32)] * 3
    else:
        out_specs = [pl.BlockSpec((RB, D), row)]
        out_shape = [jax.ShapeDtypeStruct((N, D), jnp.float32)]
    return pl.pallas_call(
        _make_post_body(with_prep),
        grid=(N // RB,),
        in_specs=in_specs,
        out_specs=out_specs,
        out_shape=out_shape,
    )(*args)


def _make_edge_kernel(chunks, lane_idx):
    perw = chunks * K
    rpt = NACC // NS
    mesh = plsc.VectorSubcoreMesh(core_axis_name="c", subcore_axis_name="s",
                                  num_cores=NC, num_subcores=NS)

    @functools.partial(
        pl.kernel,
        out_type=[jax.ShapeDtypeStruct((NC, NACC, D), jnp.float32),
                  jax.ShapeDtypeStruct((NC, NACC, L), jnp.float32)],
        mesh=mesh,
        compiler_params=pltpu.CompilerParams(use_tc_tiling_on_sc=False),
        scratch_types=[
            pltpu.VMEM((K,), jnp.int32),
            pltpu.VMEM((K,), jnp.int32),
            pltpu.VMEM((K, D), jnp.float32),
            pltpu.VMEM((K, L), jnp.float32),
            pltpu.VMEM((K, L), jnp.float32),
            pltpu.VMEM((K, D), jnp.float32),
            pltpu.VMEM((K, L), jnp.float32),
            pltpu.VMEM_SHARED((NACC, D), jnp.float32),
            pltpu.VMEM_SHARED((NACC, L), jnp.float32),
            pltpu.SemaphoreType.DMA,
            pltpu.SemaphoreType.DMA,
            pltpu.SemaphoreType.DMA,
        ],
    )
    def edge_kernel(h_hbm, as_hbm, ad_hbm, src_hbm, dst_hbm, zn_hbm, zd_hbm,
                    numer_hbm, denom_hbm,
                    srcb, dstb, hb, asb, adb, wb, eeb,
                    nacc, dacc, sem_h, sem_a, sem_d):
        c = lax.axis_index("c")
        s = lax.axis_index("s")
        wid = c * NS + s

        pltpu.sync_copy(zn_hbm, nacc.at[pl.ds(s * rpt, rpt)])
        pltpu.sync_copy(zd_hbm, dacc.at[pl.ds(s * rpt, rpt)])
        plsc.subcore_barrier()

        ebase = wid * perw

        def chunk(i, carry):
            base = ebase + i * K
            pltpu.sync_copy(src_hbm.at[pl.ds(base, K)], srcb)
            pltpu.sync_copy(dst_hbm.at[pl.ds(base, K)], dstb)
            cph = pltpu.async_copy(h_hbm.at[srcb], hb, sem_h)
            cpa = pltpu.async_copy(as_hbm.at[srcb], asb, sem_a)
            cpd = pltpu.async_copy(ad_hbm.at[dstb], adb, sem_d)
            cph.wait()
            cpa.wait()
            cpd.wait()
            for k in range(K):
                e = asb[k] + adb[k]
                e = jnp.maximum(e, 0.2 * e)
                ee = jnp.exp(e)
                eeb[k] = ee
                for j in range(D // L):
                    eej = jnp.broadcast_to(ee[lane_idx[j]], (L,))
                    wb[k, pl.ds(j * L, L)] = hb[k, pl.ds(j * L, L)] * eej
            pltpu.sync_copy(wb, nacc.at[dstb], add=True)
            pltpu.sync_copy(eeb, dacc.at[dstb], add=True)
            return carry

        lax.fori_loop(0, chunks, chunk, 0)
        plsc.subcore_barrier()

        pltpu.sync_copy(nacc.at[pl.ds(s * rpt, rpt)],
                        numer_hbm.at[c, pl.ds(s * rpt, rpt)])
        pltpu.sync_copy(dacc.at[pl.ds(s * rpt, rpt)],
                        denom_hbm.at[c, pl.ds(s * rpt, rpt)])

    return edge_kernel


def _sd_mat(att_s, att_d):
    och = att_s.shape[1]
    rows = jnp.arange(D, dtype=jnp.int32)
    cols = rows // och
    m = jnp.zeros((D, D), jnp.float32)
    m = m.at[rows, cols].set(att_s.reshape(-1))
    m = m.at[rows, L + cols].set(att_d.reshape(-1))
    return m


def _r_mat(heads, och):
    r = np.zeros((L, D), np.float32)
    for j in range(heads):
        r[j, j * och:(j + 1) * och] = 1.0
    return jnp.asarray(r)


def kernel(x, edge_index, W0, as0, ad0, b0, g0, be0,
           W1, as1, ad1, b1, g1, be1, W2, as2, ad2, b2, g2, be2):
    ei = edge_index.astype(jnp.int32)
    loop = jnp.arange(N, dtype=jnp.int32)
    src = jnp.concatenate([ei[0], loop])
    dst = jnp.concatenate([ei[1], loop])
    et = src.shape[0]
    chunks = -(-et // (NW * K))
    epad = NW * K * chunks
    padn = epad - et
    pad_idx = jnp.arange(padn, dtype=jnp.int32)
    srcp = jnp.concatenate([src, pad_idx % 16])
    dstp = jnp.concatenate([dst, N + pad_idx % (NACC - N)])
    zn = jnp.zeros((NACC // NS, D), jnp.float32)
    zd = jnp.zeros((NACC // NS, L), jnp.float32)

    edge8 = _make_edge_kernel(chunks, tuple(range(8)))
    edge1 = _make_edge_kernel(chunks, (0,) * 8)
    r8 = _r_mat(8, 16)
    r1 = _r_mat(1, D)

    h0p, asd0 = _tc_prep(x, W0, _sd_mat(as0, ad0))
    num0, den0 = edge8(h0p, asd0[:, :L], asd0[:, L:2 * L], srcp, dstp, zn, zd)
    h1, h1p, asd1 = _tc_post(x, num0[0, :N], num0[1, :N],
                             den0[0, :N], den0[1, :N], r8, b0, g0, be0,
                             W1, _sd_mat(as1, ad1))
    num1, den1 = edge8(h1p, asd1[:, :L], asd1[:, L:2 * L], srcp, dstp, zn, zd)
    h2, h2p, asd2 = _tc_post(h1, num1[0, :N], num1[1, :N],
                             den1[0, :N], den1[1, :N], r8, b1, g1, be1,
                             W2, _sd_mat(as2, ad2))
    num2, den2 = edge1(h2p, asd2[:, :L], asd2[:, L:2 * L], srcp, dstp, zn, zd)
    (h3,) = _tc_post(h2, num2[0, :N], num2[1, :N],
                     den2[0, :N], den2[1, :N], r1, b2, g2, be2)
    return h3


# speedup vs baseline: 106.6570x; 1.8306x over previous
import functools

import numpy as np
import jax
import jax.numpy as jnp
from jax import lax
from jax.experimental import pallas as pl
from jax.experimental.pallas import tpu as pltpu
from jax.experimental.pallas import tpu_sc as plsc

N = 10000
D = 128
L = 16
NC = 2
NS = 16
NW = NC * NS
K = 64
NACC = 10112
RB = 400


def _prep_body(h_ref, w_ref, sd_ref, hp_ref, asd_ref):
    hp = jnp.dot(h_ref[...], w_ref[...], preferred_element_type=jnp.float32)
    hp_ref[...] = hp
    asd_ref[...] = jnp.dot(hp, sd_ref[...], preferred_element_type=jnp.float32)


def _tc_prep(h, W, SD):
    return pl.pallas_call(
        _prep_body,
        grid=(N // RB,),
        in_specs=[
            pl.BlockSpec((RB, D), lambda i: (i, 0)),
            pl.BlockSpec((D, D), lambda i: (0, 0)),
            pl.BlockSpec((D, D), lambda i: (0, 0)),
        ],
        out_specs=[pl.BlockSpec((RB, D), lambda i: (i, 0))] * 2,
        out_shape=[jax.ShapeDtypeStruct((N, D), jnp.float32)] * 2,
    )(h, W, SD)


def _make_post_body(with_prep):
    def body(hprev_ref, n0_ref, n1_ref, d0_ref, d1_ref, rm_ref, b_ref,
             g_ref, be_ref, *rest):
        if with_prep:
            w_ref, sd_ref, hn_ref, hp_ref, asd_ref = rest
        else:
            (hn_ref,) = rest
        den = d0_ref[...] + d1_ref[...]
        den_exp = jnp.dot(den, rm_ref[...], preferred_element_type=jnp.float32)
        gat = (n0_ref[...] + n1_ref[...]) / (den_exp + 1e-16) + b_ref[...]
        mu = jnp.mean(gat, axis=-1, keepdims=True)
        xc = gat - mu
        var = jnp.mean(xc * xc, axis=-1, keepdims=True)
        y = xc * lax.rsqrt(var + 1e-5) * g_ref[...] + be_ref[...]
        z = hprev_ref[...] + y
        hn = jnp.where(z > 0, z, jnp.exp(jnp.minimum(z, 0.0)) - 1.0)
        hn_ref[...] = hn
        if with_prep:
            hp = jnp.dot(hn, w_ref[...], preferred_element_type=jnp.float32)
            hp_ref[...] = hp
            asd_ref[...] = jnp.dot(hp, sd_ref[...],
                                   preferred_element_type=jnp.float32)
    return body


def _tc_post(hprev, n0, n1, d0, d1, Rm, b, g, be, W=None, SD=None):
    with_prep = W is not None
    full = lambda i: (0, 0)
    row = lambda i: (i, 0)
    in_specs = [
        pl.BlockSpec((RB, D), row),
        pl.BlockSpec((RB, D), row),
        pl.BlockSpec((RB, D), row),
        pl.BlockSpec((RB, L), row),
        pl.BlockSpec((RB, L), row),
        pl.BlockSpec((L, D), full),
        pl.BlockSpec((1, D), full),
        pl.BlockSpec((1, D), full),
        pl.BlockSpec((1, D), full),
    ]
    args = [hprev, n0, n1, d0, d1, Rm, b.reshape(1, D), g.reshape(1, D),
            be.reshape(1, D)]
    if with_prep:
        in_specs += [pl.BlockSpec((D, D), full), pl.BlockSpec((D, D), full)]
        args += [W, SD]
        out_specs = [pl.BlockSpec((RB, D), row)] * 3
        out_shape = [jax.ShapeDtypeStruct((N, D), jnp.float32)] * 3
    else:
        out_specs = [pl.BlockSpec((RB, D), row)]
        out_shape = [jax.ShapeDtypeStruct((N, D), jnp.float32)]
    return pl.pallas_call(
        _make_post_body(with_prep),
        grid=(N // RB,),
        in_specs=in_specs,
        out_specs=out_specs,
        out_shape=out_shape,
    )(*args)


def _make_edge_kernel(chunks, lane_idx):
    assert chunks % 2 == 0
    perw = chunks * K
    rpt = NACC // NS
    mesh = plsc.VectorSubcoreMesh(core_axis_name="c", subcore_axis_name="s",
                                  num_cores=NC, num_subcores=NS)

    @functools.partial(
        pl.kernel,
        out_type=[jax.ShapeDtypeStruct((NC, NACC, D), jnp.float32),
                  jax.ShapeDtypeStruct((NC, NACC, L), jnp.float32)],
        mesh=mesh,
        compiler_params=pltpu.CompilerParams(use_tc_tiling_on_sc=False),
        scratch_types=[
            pltpu.VMEM((K,), jnp.int32), pltpu.VMEM((K,), jnp.int32),
            pltpu.VMEM((K,), jnp.int32), pltpu.VMEM((K,), jnp.int32),
            pltpu.VMEM((K,), jnp.int32), pltpu.VMEM((K,), jnp.int32),
            pltpu.VMEM((K, D), jnp.float32), pltpu.VMEM((K, D), jnp.float32),
            pltpu.VMEM((K, L), jnp.float32), pltpu.VMEM((K, L), jnp.float32),
            pltpu.VMEM((K, L), jnp.float32), pltpu.VMEM((K, L), jnp.float32),
            pltpu.VMEM((K, D), jnp.float32), pltpu.VMEM((K, D), jnp.float32),
            pltpu.VMEM((K, L), jnp.float32), pltpu.VMEM((K, L), jnp.float32),
            pltpu.VMEM_SHARED((NACC, D), jnp.float32),
            pltpu.VMEM_SHARED((NACC, L), jnp.float32),
            pltpu.SemaphoreType.DMA,
            pltpu.SemaphoreType.DMA, pltpu.SemaphoreType.DMA,
            pltpu.SemaphoreType.DMA, pltpu.SemaphoreType.DMA,
        ],
    )
    def edge_kernel(h_hbm, as_hbm, ad_hbm, src_hbm, dst_hbm, zn_hbm, zd_hbm,
                    numer_hbm, denom_hbm,
                    srcb0, srcb1, dstb0, dstb1, sdst0, sdst1,
                    hb0, hb1, asb0, asb1, adb0, adb1,
                    wb0, wb1, eeb0, eeb1,
                    nacc, dacc, sem_g, sem_i0, sem_i1, sem_s0, sem_s1):
        srcb = (srcb0, srcb1)
        dstb = (dstb0, dstb1)
        sdst = (sdst0, sdst1)
        hb = (hb0, hb1)
        asb = (asb0, asb1)
        adb = (adb0, adb1)
        wb = (wb0, wb1)
        eeb = (eeb0, eeb1)
        sem_i = (sem_i0, sem_i1)
        sem_s = (sem_s0, sem_s1)

        c = lax.axis_index("c")
        s = lax.axis_index("s")
        wid = c * NS + s
        ebase = wid * perw

        pltpu.sync_copy(zn_hbm, nacc.at[pl.ds(s * rpt, rpt)])
        pltpu.sync_copy(zd_hbm, dacc.at[pl.ds(s * rpt, rpt)])
        plsc.subcore_barrier()

        def idx_issue(ci, b):
            base = ebase + ci * K
            pltpu.async_copy(src_hbm.at[pl.ds(base, K)], srcb[b], sem_i[b])
            pltpu.async_copy(dst_hbm.at[pl.ds(base, K)], dstb[b], sem_i[b])

        def idx_wait(b):
            pltpu.make_async_copy(src_hbm.at[pl.ds(0, K)], srcb[b],
                                  sem_i[b]).wait()
            pltpu.make_async_copy(dst_hbm.at[pl.ds(0, K)], dstb[b],
                                  sem_i[b]).wait()

        def gather_issue(b):
            pltpu.async_copy(h_hbm.at[srcb[b]], hb[b], sem_g)
            pltpu.async_copy(as_hbm.at[srcb[b]], asb[b], sem_g)
            pltpu.async_copy(ad_hbm.at[dstb[b]], adb[b], sem_g)

        def gather_wait(b):
            pltpu.make_async_copy(h_hbm.at[srcb[b]], hb[b], sem_g).wait()
            pltpu.make_async_copy(as_hbm.at[srcb[b]], asb[b], sem_g).wait()
            pltpu.make_async_copy(ad_hbm.at[dstb[b]], adb[b], sem_g).wait()

        def scatter_issue(b):
            pltpu.async_copy(wb[b], nacc.at[sdst[b]], sem_s[b], add=True)
            pltpu.async_copy(eeb[b], dacc.at[sdst[b]], sem_s[b], add=True)

        def scatter_drain(b):
            pltpu.make_async_copy(wb[b], nacc.at[sdst[b]], sem_s[b]).wait()
            pltpu.make_async_copy(eeb[b], dacc.at[sdst[b]], sem_s[b]).wait()

        def compute(b):
            ublanes = sorted(set(lane_idx))
            for k in range(K):
                e = asb[b][k] + adb[b][k]
                e = jnp.maximum(e, 0.2 * e)
                ee = jnp.exp(e)
                eeb[b][k] = ee
                bc = {ln: jnp.broadcast_to(ee[ln], (L,)) for ln in ublanes}
                for j in range(D // L):
                    wb[b][k, pl.ds(j * L, L)] = (
                        hb[b][k, pl.ds(j * L, L)] * bc[lane_idx[j]])

        idx_issue(0, 0)
        idx_issue(1, 1)
        idx_wait(0)
        gather_issue(0)

        def pair(j, carry):
            for b in (0, 1):
                ci = 2 * j + b
                gather_wait(b)

                @pl.when(ci >= 2)
                def _():
                    scatter_drain(b)

                for r in range(K // L):
                    sdst[b][pl.ds(r * L, L)] = dstb[b][pl.ds(r * L, L)]

                @pl.when(ci + 1 < chunks)
                def _():
                    idx_wait(1 - b)
                    gather_issue(1 - b)

                @pl.when(ci + 2 < chunks)
                def _():
                    idx_issue(ci + 2, b)

                compute(b)
                scatter_issue(b)
            return carry

        lax.fori_loop(0, chunks // 2, pair, 0)
        scatter_drain(0)
        scatter_drain(1)
        plsc.subcore_barrier()

        pltpu.sync_copy(nacc.at[pl.ds(s * rpt, rpt)],
                        numer_hbm.at[c, pl.ds(s * rpt, rpt)])
        pltpu.sync_copy(dacc.at[pl.ds(s * rpt, rpt)],
                        denom_hbm.at[c, pl.ds(s * rpt, rpt)])

    return edge_kernel


def _sd_mat(att_s, att_d):
    och = att_s.shape[1]
    rows = jnp.arange(D, dtype=jnp.int32)
    cols = rows // och
    m = jnp.zeros((D, D), jnp.float32)
    m = m.at[rows, cols].set(att_s.reshape(-1))
    m = m.at[rows, L + cols].set(att_d.reshape(-1))
    return m


def _r_mat(heads, och):
    r = np.zeros((L, D), np.float32)
    for j in range(heads):
        r[j, j * och:(j + 1) * och] = 1.0
    return jnp.asarray(r)


def kernel(x, edge_index, W0, as0, ad0, b0, g0, be0,
           W1, as1, ad1, b1, g1, be1, W2, as2, ad2, b2, g2, be2):
    ei = edge_index.astype(jnp.int32)
    loop = jnp.arange(N, dtype=jnp.int32)
    src = jnp.concatenate([ei[0], loop])
    dst = jnp.concatenate([ei[1], loop])
    et = src.shape[0]
    chunks = 2 * -(-et // (NW * K * 2))
    epad = NW * K * chunks
    padn = epad - et
    pad_idx = jnp.arange(padn, dtype=jnp.int32)
    srcp = jnp.concatenate([src, pad_idx % 16])
    dstp = jnp.concatenate([dst, N + pad_idx % (NACC - N)])
    zn = jnp.zeros((NACC // NS, D), jnp.float32)
    zd = jnp.zeros((NACC // NS, L), jnp.float32)

    edge8 = _make_edge_kernel(chunks, tuple(range(8)))
    edge1 = _make_edge_kernel(chunks, (0,) * 8)
    r8 = _r_mat(8, 16)
    r1 = _r_mat(1, D)

    h0p, asd0 = _tc_prep(x, W0, _sd_mat(as0, ad0))
    num0, den0 = edge8(h0p, asd0[:, :L], asd0[:, L:2 * L], srcp, dstp, zn, zd)
    h1, h1p, asd1 = _tc_post(x, num0[0, :N], num0[1, :N],
                             den0[0, :N], den0[1, :N], r8, b0, g0, be0,
                             W1, _sd_mat(as1, ad1))
    num1, den1 = edge8(h1p, asd1[:, :L], asd1[:, L:2 * L], srcp, dstp, zn, zd)
    h2, h2p, asd2 = _tc_post(h1, num1[0, :N], num1[1, :N],
                             den1[0, :N], den1[1, :N], r8, b1, g1, be1,
                             W2, _sd_mat(as2, ad2))
    num2, den2 = edge1(h2p, asd2[:, :L], asd2[:, L:2 * L], srcp, dstp, zn, zd)
    (h3,) = _tc_post(h2, num2[0, :N], num2[1, :N],
                     den2[0, :N], den2[1, :N], r1, b2, g2, be2)
    return h3


# speedup vs baseline: 113.9398x; 1.0683x over previous
import functools

import numpy as np
import jax
import jax.numpy as jnp
from jax import lax
from jax.experimental import pallas as pl
from jax.experimental.pallas import tpu as pltpu
from jax.experimental.pallas import tpu_sc as plsc

N = 10000
D = 128
L = 16
NC = 2
NS = 16
NW = NC * NS
K = 64
NACC = 10112
RB = 400


def _prep_body(h_ref, w_ref, sd_ref, hp_ref, as_ref, ad_ref):
    hp = jnp.dot(h_ref[...], w_ref[...], preferred_element_type=jnp.float32)
    hp_ref[...] = hp
    asd = jnp.dot(hp, sd_ref[...], preferred_element_type=jnp.float32)
    as_ref[...] = asd[:, :L]
    ad_ref[...] = asd[:, L:2 * L]


def _tc_prep(h, W, SD):
    return pl.pallas_call(
        _prep_body,
        grid=(N // RB,),
        in_specs=[
            pl.BlockSpec((RB, D), lambda i: (i, 0)),
            pl.BlockSpec((D, D), lambda i: (0, 0)),
            pl.BlockSpec((D, D), lambda i: (0, 0)),
        ],
        out_specs=[pl.BlockSpec((RB, D), lambda i: (i, 0)),
                   pl.BlockSpec((RB, L), lambda i: (i, 0)),
                   pl.BlockSpec((RB, L), lambda i: (i, 0))],
        out_shape=[jax.ShapeDtypeStruct((N, D), jnp.float32),
                   jax.ShapeDtypeStruct((N, L), jnp.float32),
                   jax.ShapeDtypeStruct((N, L), jnp.float32)],
    )(h, W, SD)


def _make_post_body(with_prep):
    def body(hprev_ref, num_ref, den_ref, rm_ref, b_ref,
             g_ref, be_ref, *rest):
        if with_prep:
            w_ref, sd_ref, hn_ref, hp_ref, as_ref, ad_ref = rest
        else:
            (hn_ref,) = rest
        den = den_ref[0] + den_ref[1]
        den_exp = jnp.dot(den, rm_ref[...], preferred_element_type=jnp.float32)
        gat = (num_ref[0] + num_ref[1]) / (den_exp + 1e-16) + b_ref[...]
        mu = jnp.mean(gat, axis=-1, keepdims=True)
        xc = gat - mu
        var = jnp.mean(xc * xc, axis=-1, keepdims=True)
        y = xc * lax.rsqrt(var + 1e-5) * g_ref[...] + be_ref[...]
        z = hprev_ref[...] + y
        hn = jnp.where(z > 0, z, jnp.exp(jnp.minimum(z, 0.0)) - 1.0)
        hn_ref[...] = hn
        if with_prep:
            hp = jnp.dot(hn, w_ref[...], preferred_element_type=jnp.float32)
            hp_ref[...] = hp
            asd = jnp.dot(hp, sd_ref[...], preferred_element_type=jnp.float32)
            as_ref[...] = asd[:, :L]
            ad_ref[...] = asd[:, L:2 * L]
    return body


def _tc_post(hprev, num, den, Rm, b, g, be, W=None, SD=None):
    with_prep = W is not None
    full = lambda i: (0, 0)
    row = lambda i: (i, 0)
    in_specs = [
        pl.BlockSpec((RB, D), row),
        pl.BlockSpec((NC, RB, D), lambda i: (0, i, 0)),
        pl.BlockSpec((NC, RB, L), lambda i: (0, i, 0)),
        pl.BlockSpec((L, D), full),
        pl.BlockSpec((1, D), full),
        pl.BlockSpec((1, D), full),
        pl.BlockSpec((1, D), full),
    ]
    args = [hprev, num, den, Rm, b.reshape(1, D), g.reshape(1, D),
            be.reshape(1, D)]
    if with_prep:
        in_specs += [pl.BlockSpec((D, D), full), pl.BlockSpec((D, D), full)]
        args += [W, SD]
        out_specs = [pl.BlockSpec((RB, D), row)] * 2 + \
                    [pl.BlockSpec((RB, L), row)] * 2
        out_shape = [jax.ShapeDtypeStruct((N, D), jnp.float32)] * 2 + \
                    [jax.ShapeDtypeStruct((N, L), jnp.float32)] * 2
    else:
        out_specs = [pl.BlockSpec((RB, D), row)]
        out_shape = [jax.ShapeDtypeStruct((N, D), jnp.float32)]
    return pl.pallas_call(
        _make_post_body(with_prep),
        grid=(N // RB,),
        in_specs=in_specs,
        out_specs=out_specs,
        out_shape=out_shape,
    )(*args)


def _make_edge_kernel(chunks, lane_idx):
    assert chunks % 2 == 0
    perw = chunks * K
    rpt = NACC // NS
    mesh = plsc.VectorSubcoreMesh(core_axis_name="c", subcore_axis_name="s",
                                  num_cores=NC, num_subcores=NS)

    @functools.partial(
        pl.kernel,
        out_type=[jax.ShapeDtypeStruct((NC, N, D), jnp.float32),
                  jax.ShapeDtypeStruct((NC, N, L), jnp.float32)],
        mesh=mesh,
        compiler_params=pltpu.CompilerParams(use_tc_tiling_on_sc=False),
        scratch_types=[
            pltpu.VMEM((K,), jnp.int32), pltpu.VMEM((K,), jnp.int32),
            pltpu.VMEM((K,), jnp.int32), pltpu.VMEM((K,), jnp.int32),
            pltpu.VMEM((K,), jnp.int32), pltpu.VMEM((K,), jnp.int32),
            pltpu.VMEM((K, D), jnp.float32), pltpu.VMEM((K, D), jnp.float32),
            pltpu.VMEM((K, L), jnp.float32), pltpu.VMEM((K, L), jnp.float32),
            pltpu.VMEM((K, L), jnp.float32), pltpu.VMEM((K, L), jnp.float32),
            pltpu.VMEM((K, D), jnp.float32), pltpu.VMEM((K, D), jnp.float32),
            pltpu.VMEM((K, L), jnp.float32), pltpu.VMEM((K, L), jnp.float32),
            pltpu.VMEM_SHARED((NACC, D), jnp.float32),
            pltpu.VMEM_SHARED((NACC, L), jnp.float32),
            pltpu.SemaphoreType.DMA,
            pltpu.SemaphoreType.DMA, pltpu.SemaphoreType.DMA,
            pltpu.SemaphoreType.DMA, pltpu.SemaphoreType.DMA,
        ],
    )
    def edge_kernel(h_hbm, as_hbm, ad_hbm, src_hbm, dst_hbm, zn_hbm, zd_hbm,
                    numer_hbm, denom_hbm,
                    srcb0, srcb1, dstb0, dstb1, sdst0, sdst1,
                    hb0, hb1, asb0, asb1, adb0, adb1,
                    wb0, wb1, eeb0, eeb1,
                    nacc, dacc, sem_g, sem_i0, sem_i1, sem_s0, sem_s1):
        srcb = (srcb0, srcb1)
        dstb = (dstb0, dstb1)
        sdst = (sdst0, sdst1)
        hb = (hb0, hb1)
        asb = (asb0, asb1)
        adb = (adb0, adb1)
        wb = (wb0, wb1)
        eeb = (eeb0, eeb1)
        sem_i = (sem_i0, sem_i1)
        sem_s = (sem_s0, sem_s1)

        c = lax.axis_index("c")
        s = lax.axis_index("s")
        wid = c * NS + s
        ebase = wid * perw

        pltpu.sync_copy(zn_hbm, nacc.at[pl.ds(s * rpt, rpt)])
        pltpu.sync_copy(zd_hbm, dacc.at[pl.ds(s * rpt, rpt)])
        plsc.subcore_barrier()

        def idx_issue(ci, b):
            base = ebase + ci * K
            pltpu.async_copy(src_hbm.at[pl.ds(base, K)], srcb[b], sem_i[b])
            pltpu.async_copy(dst_hbm.at[pl.ds(base, K)], dstb[b], sem_i[b])

        def idx_wait(b):
            pltpu.make_async_copy(src_hbm.at[pl.ds(0, K)], srcb[b],
                                  sem_i[b]).wait()
            pltpu.make_async_copy(dst_hbm.at[pl.ds(0, K)], dstb[b],
                                  sem_i[b]).wait()

        def gather_issue(b):
            pltpu.async_copy(h_hbm.at[srcb[b]], hb[b], sem_g)
            pltpu.async_copy(as_hbm.at[srcb[b]], asb[b], sem_g)
            pltpu.async_copy(ad_hbm.at[dstb[b]], adb[b], sem_g)

        def gather_wait(b):
            pltpu.make_async_copy(h_hbm.at[srcb[b]], hb[b], sem_g).wait()
            pltpu.make_async_copy(as_hbm.at[srcb[b]], asb[b], sem_g).wait()
            pltpu.make_async_copy(ad_hbm.at[dstb[b]], adb[b], sem_g).wait()

        def scatter_issue(b):
            pltpu.async_copy(wb[b], nacc.at[sdst[b]], sem_s[b], add=True)
            pltpu.async_copy(eeb[b], dacc.at[sdst[b]], sem_s[b], add=True)

        def scatter_drain(b):
            pltpu.make_async_copy(wb[b], nacc.at[sdst[b]], sem_s[b]).wait()
            pltpu.make_async_copy(eeb[b], dacc.at[sdst[b]], sem_s[b]).wait()

        def compute(b):
            ublanes = sorted(set(lane_idx))
            for k in range(K):
                e = asb[b][k] + adb[b][k]
                e = jnp.maximum(e, 0.2 * e)
                ee = jnp.exp(e)
                eeb[b][k] = ee
                bc = {ln: jnp.broadcast_to(ee[ln], (L,)) for ln in ublanes}
                for j in range(D // L):
                    wb[b][k, pl.ds(j * L, L)] = (
                        hb[b][k, pl.ds(j * L, L)] * bc[lane_idx[j]])

        idx_issue(0, 0)
        idx_issue(1, 1)
        idx_wait(0)
        gather_issue(0)

        def pair(j, carry):
            for b in (0, 1):
                ci = 2 * j + b
                gather_wait(b)

                @pl.when(ci >= 2)
                def _():
                    scatter_drain(b)

                for r in range(K // L):
                    sdst[b][pl.ds(r * L, L)] = dstb[b][pl.ds(r * L, L)]

                @pl.when(ci + 1 < chunks)
                def _():
                    idx_wait(1 - b)
                    gather_issue(1 - b)

                @pl.when(ci + 2 < chunks)
                def _():
                    idx_issue(ci + 2, b)

                compute(b)
                scatter_issue(b)
            return carry

        lax.fori_loop(0, chunks // 2, pair, 0)
        scatter_drain(0)
        scatter_drain(1)
        plsc.subcore_barrier()

        rout = N // NS
        pltpu.sync_copy(nacc.at[pl.ds(s * rout, rout)],
                        numer_hbm.at[c, pl.ds(s * rout, rout)])
        pltpu.sync_copy(dacc.at[pl.ds(s * rout, rout)],
                        denom_hbm.at[c, pl.ds(s * rout, rout)])

    return edge_kernel


def _sd_mat(att_s, att_d):
    och = att_s.shape[1]
    rows = jnp.arange(D, dtype=jnp.int32)
    cols = rows // och
    m = jnp.zeros((D, D), jnp.float32)
    m = m.at[rows, cols].set(att_s.reshape(-1))
    m = m.at[rows, L + cols].set(att_d.reshape(-1))
    return m


def _r_mat(heads, och):
    r = np.zeros((L, D), np.float32)
    for j in range(heads):
        r[j, j * och:(j + 1) * och] = 1.0
    return jnp.asarray(r)


def kernel(x, edge_index, W0, as0, ad0, b0, g0, be0,
           W1, as1, ad1, b1, g1, be1, W2, as2, ad2, b2, g2, be2):
    ei = edge_index.astype(jnp.int32)
    loop = jnp.arange(N, dtype=jnp.int32)
    src = jnp.concatenate([ei[0], loop])
    dst = jnp.concatenate([ei[1], loop])
    et = src.shape[0]
    chunks = 2 * -(-et // (NW * K * 2))
    epad = NW * K * chunks
    padn = epad - et
    pad_idx = jnp.arange(padn, dtype=jnp.int32)
    srcp = jnp.concatenate([src, pad_idx % 16])
    dstp = jnp.concatenate([dst, N + pad_idx % (NACC - N)])
    zn = jnp.zeros((NACC // NS, D), jnp.float32)
    zd = jnp.zeros((NACC // NS, L), jnp.float32)

    edge8 = _make_edge_kernel(chunks, tuple(range(8)))
    edge1 = _make_edge_kernel(chunks, (0,) * 8)
    r8 = _r_mat(8, 16)
    r1 = _r_mat(1, D)

    h0p, as_0, ad_0 = _tc_prep(x, W0, _sd_mat(as0, ad0))
    num0, den0 = edge8(h0p, as_0, ad_0, srcp, dstp, zn, zd)
    h1, h1p, as_1, ad_1 = _tc_post(x, num0, den0, r8, b0, g0, be0,
                                   W1, _sd_mat(as1, ad1))
    num1, den1 = edge8(h1p, as_1, ad_1, srcp, dstp, zn, zd)
    h2, h2p, as_2, ad_2 = _tc_post(h1, num1, den1, r8, b1, g1, be1,
                                   W2, _sd_mat(as2, ad2))
    num2, den2 = edge1(h2p, as_2, ad_2, srcp, dstp, zn, zd)
    (h3,) = _tc_post(h2, num2, den2, r1, b2, g2, be2)
    return h3


# speedup vs baseline: 116.0919x; 1.0189x over previous
import functools

import numpy as np
import jax
import jax.numpy as jnp
from jax import lax
from jax.experimental import pallas as pl
from jax.experimental.pallas import tpu as pltpu
from jax.experimental.pallas import tpu_sc as plsc

N = 10000
D = 128
L = 16
NC = 2
NS = 16
NW = NC * NS
K = 48
NACC = 10016
RB = 400


def _prep_body(h_ref, w_ref, sd_ref, hp_ref, as_ref, ad_ref):
    hp = jnp.dot(h_ref[...], w_ref[...], preferred_element_type=jnp.float32)
    hp_ref[...] = hp
    asd = jnp.dot(hp, sd_ref[...], preferred_element_type=jnp.float32)
    as_ref[...] = asd[:, :L]
    ad_ref[...] = asd[:, L:2 * L]


def _tc_prep(h, W, SD):
    return pl.pallas_call(
        _prep_body,
        grid=(N // RB,),
        in_specs=[
            pl.BlockSpec((RB, D), lambda i: (i, 0)),
            pl.BlockSpec((D, D), lambda i: (0, 0)),
            pl.BlockSpec((D, D), lambda i: (0, 0)),
        ],
        out_specs=[pl.BlockSpec((RB, D), lambda i: (i, 0)),
                   pl.BlockSpec((RB, L), lambda i: (i, 0)),
                   pl.BlockSpec((RB, L), lambda i: (i, 0))],
        out_shape=[jax.ShapeDtypeStruct((N, D), jnp.float32),
                   jax.ShapeDtypeStruct((N, L), jnp.float32),
                   jax.ShapeDtypeStruct((N, L), jnp.float32)],
    )(h, W, SD)


def _make_post_body(with_prep):
    def body(hprev_ref, num_ref, den_ref, rm_ref, b_ref,
             g_ref, be_ref, *rest):
        if with_prep:
            w_ref, sd_ref, hn_ref, hp_ref, as_ref, ad_ref = rest
        else:
            (hn_ref,) = rest
        den = den_ref[0] + den_ref[1]
        den_exp = jnp.dot(den, rm_ref[...], preferred_element_type=jnp.float32)
        gat = (num_ref[0] + num_ref[1]) / (den_exp + 1e-16) + b_ref[...]
        mu = jnp.mean(gat, axis=-1, keepdims=True)
        xc = gat - mu
        var = jnp.mean(xc * xc, axis=-1, keepdims=True)
        y = xc * lax.rsqrt(var + 1e-5) * g_ref[...] + be_ref[...]
        z = hprev_ref[...] + y
        hn = jnp.where(z > 0, z, jnp.exp(jnp.minimum(z, 0.0)) - 1.0)
        hn_ref[...] = hn
        if with_prep:
            hp = jnp.dot(hn, w_ref[...], preferred_element_type=jnp.float32)
            hp_ref[...] = hp
            asd = jnp.dot(hp, sd_ref[...], preferred_element_type=jnp.float32)
            as_ref[...] = asd[:, :L]
            ad_ref[...] = asd[:, L:2 * L]
    return body


def _tc_post(hprev, num, den, Rm, b, g, be, W=None, SD=None):
    with_prep = W is not None
    full = lambda i: (0, 0)
    row = lambda i: (i, 0)
    in_specs = [
        pl.BlockSpec((RB, D), row),
        pl.BlockSpec((NC, RB, D), lambda i: (0, i, 0)),
        pl.BlockSpec((NC, RB, L), lambda i: (0, i, 0)),
        pl.BlockSpec((L, D), full),
        pl.BlockSpec((1, D), full),
        pl.BlockSpec((1, D), full),
        pl.BlockSpec((1, D), full),
    ]
    args = [hprev, num, den, Rm, b.reshape(1, D), g.reshape(1, D),
            be.reshape(1, D)]
    if with_prep:
        in_specs += [pl.BlockSpec((D, D), full), pl.BlockSpec((D, D), full)]
        args += [W, SD]
        out_specs = [pl.BlockSpec((RB, D), row)] * 2 + \
                    [pl.BlockSpec((RB, L), row)] * 2
        out_shape = [jax.ShapeDtypeStruct((N, D), jnp.float32)] * 2 + \
                    [jax.ShapeDtypeStruct((N, L), jnp.float32)] * 2
    else:
        out_specs = [pl.BlockSpec((RB, D), row)]
        out_shape = [jax.ShapeDtypeStruct((N, D), jnp.float32)]
    return pl.pallas_call(
        _make_post_body(with_prep),
        grid=(N // RB,),
        in_specs=in_specs,
        out_specs=out_specs,
        out_shape=out_shape,
    )(*args)


def _make_edge_kernel(chunks, lane_idx):
    NB = 4
    assert chunks % NB == 0
    perw = chunks * K
    rpt = NACC // NS
    mesh = plsc.VectorSubcoreMesh(core_axis_name="c", subcore_axis_name="s",
                                  num_cores=NC, num_subcores=NS)

    @functools.partial(
        pl.kernel,
        out_type=[jax.ShapeDtypeStruct((NC, N, D), jnp.float32),
                  jax.ShapeDtypeStruct((NC, N, L), jnp.float32)],
        mesh=mesh,
        compiler_params=pltpu.CompilerParams(use_tc_tiling_on_sc=False),
        scratch_types=(
            [pltpu.VMEM((K,), jnp.int32)] * (4 * NB) +
            [pltpu.VMEM((K, D), jnp.float32)] * NB +
            [pltpu.VMEM((K, L), jnp.float32)] * (3 * NB) +
            [pltpu.VMEM_SHARED((NACC, D), jnp.float32),
             pltpu.VMEM_SHARED((NACC, L), jnp.float32)] +
            [pltpu.SemaphoreType.DMA] * (3 * NB)
        ),
    )
    def edge_kernel(h_hbm, as_hbm, ad_hbm, pk_hbm, zn_hbm, zd_hbm,
                    numer_hbm, denom_hbm, *scratch):
        pkb = scratch[0:NB]
        srcb = scratch[NB:2 * NB]
        dstb = scratch[2 * NB:3 * NB]
        sdst = scratch[3 * NB:4 * NB]
        hb = scratch[4 * NB:5 * NB]
        asb = scratch[5 * NB:6 * NB]
        adb = scratch[6 * NB:7 * NB]
        eeb = scratch[7 * NB:8 * NB]
        nacc = scratch[8 * NB]
        dacc = scratch[8 * NB + 1]
        sem_g = scratch[8 * NB + 2:8 * NB + 2 + NB]
        sem_i = scratch[8 * NB + 2 + NB:8 * NB + 2 + 2 * NB]
        sem_s = scratch[8 * NB + 2 + 2 * NB:8 * NB + 2 + 3 * NB]

        c = lax.axis_index("c")
        s = lax.axis_index("s")
        wid = c * NS + s
        ebase = wid * perw

        pltpu.sync_copy(zn_hbm, nacc.at[pl.ds(s * rpt, rpt)])
        pltpu.sync_copy(zd_hbm, dacc.at[pl.ds(s * rpt, rpt)])
        plsc.subcore_barrier()

        def idx_issue(ci, b):
            base = ebase + ci * K
            pltpu.async_copy(pk_hbm.at[pl.ds(base, K)], pkb[b], sem_i[b])

        def idx_wait(b):
            pltpu.make_async_copy(pk_hbm.at[pl.ds(0, K)], pkb[b],
                                  sem_i[b]).wait()

        def idx_unpack(b):
            for r in range(K // L):
                v = pkb[b][pl.ds(r * L, L)]
                srcb[b][pl.ds(r * L, L)] = v & jnp.int32(16383)
                dstb[b][pl.ds(r * L, L)] = lax.shift_right_logical(
                    v, jnp.int32(14))

        def gather_issue(b):
            pltpu.async_copy(h_hbm.at[srcb[b]], hb[b], sem_g[b])
            pltpu.async_copy(as_hbm.at[srcb[b]], asb[b], sem_g[b])
            pltpu.async_copy(ad_hbm.at[dstb[b]], adb[b], sem_g[b])

        def gather_wait(b):
            pltpu.make_async_copy(h_hbm.at[srcb[b]], hb[b], sem_g[b]).wait()
            pltpu.make_async_copy(as_hbm.at[srcb[b]], asb[b], sem_g[b]).wait()
            pltpu.make_async_copy(ad_hbm.at[dstb[b]], adb[b], sem_g[b]).wait()

        def scatter_issue(b):
            pltpu.async_copy(hb[b], nacc.at[sdst[b]], sem_s[b], add=True)
            pltpu.async_copy(eeb[b], dacc.at[sdst[b]], sem_s[b], add=True)

        def scatter_drain(b):
            pltpu.make_async_copy(hb[b], nacc.at[sdst[b]], sem_s[b]).wait()
            pltpu.make_async_copy(eeb[b], dacc.at[sdst[b]], sem_s[b]).wait()

        def compute(b):
            ublanes = sorted(set(lane_idx))
            for k in range(K):
                e = asb[b][k] + adb[b][k]
                e = jnp.maximum(e, 0.2 * e)
                ee = jnp.exp(e)
                eeb[b][k] = ee
                bc = {ln: jnp.broadcast_to(ee[ln], (L,)) for ln in ublanes}
                for j in range(D // L):
                    hb[b][k, pl.ds(j * L, L)] = (
                        hb[b][k, pl.ds(j * L, L)] * bc[lane_idx[j]])

        for b in range(NB):
            idx_issue(b, b)
        idx_wait(0)
        idx_unpack(0)
        gather_issue(0)
        idx_wait(1)
        idx_unpack(1)
        gather_issue(1)

        def rotation(j, carry):
            for b in range(NB):
                ci = NB * j + b
                gather_wait(b)

                for r in range(K // L):
                    sdst[b][pl.ds(r * L, L)] = dstb[b][pl.ds(r * L, L)]

                @pl.when(ci + NB < chunks)
                def _():
                    idx_issue(ci + NB, b)

                b2 = (b + 2) % NB

                @pl.when(jnp.logical_and(ci >= 2, ci + 2 < chunks))
                def _():
                    scatter_drain(b2)

                @pl.when(ci + 2 < chunks)
                def _():
                    idx_wait(b2)
                    idx_unpack(b2)
                    gather_issue(b2)

                compute(b)
                scatter_issue(b)
            return carry

        lax.fori_loop(0, chunks // NB, rotation, 0)
        scatter_drain((chunks - 4) % NB)
        scatter_drain((chunks - 3) % NB)
        scatter_drain((chunks - 2) % NB)
        scatter_drain((chunks - 1) % NB)
        plsc.subcore_barrier()

        rout = N // NS
        pltpu.sync_copy(nacc.at[pl.ds(s * rout, rout)],
                        numer_hbm.at[c, pl.ds(s * rout, rout)])
        pltpu.sync_copy(dacc.at[pl.ds(s * rout, rout)],
                        denom_hbm.at[c, pl.ds(s * rout, rout)])

    return edge_kernel


def _sd_mat(att_s, att_d):
    och = att_s.shape[1]
    rows = jnp.arange(D, dtype=jnp.int32)
    cols = rows // och
    m = jnp.zeros((D, D), jnp.float32)
    m = m.at[rows, cols].set(att_s.reshape(-1))
    m = m.at[rows, L + cols].set(att_d.reshape(-1))
    return m


def _r_mat(heads, och):
    r = np.zeros((L, D), np.float32)
    for j in range(heads):
        r[j, j * och:(j + 1) * och] = 1.0
    return jnp.asarray(r)


def kernel(x, edge_index, W0, as0, ad0, b0, g0, be0,
           W1, as1, ad1, b1, g1, be1, W2, as2, ad2, b2, g2, be2):
    ei = edge_index.astype(jnp.int32)
    loop = jnp.arange(N, dtype=jnp.int32)
    src = jnp.concatenate([ei[0], loop])
    dst = jnp.concatenate([ei[1], loop])
    et = src.shape[0]
    chunks = 4 * -(-et // (NW * K * 4))
    epad = NW * K * chunks
    padn = epad - et
    pad_idx = jnp.arange(padn, dtype=jnp.int32)
    srcp = jnp.concatenate([src, pad_idx % 16])
    dstp = jnp.concatenate([dst, N + pad_idx % (NACC - N)])
    pk = srcp | (dstp << 14)
    zn = jnp.zeros((NACC // NS, D), jnp.float32)
    zd = jnp.zeros((NACC // NS, L), jnp.float32)

    edge8 = _make_edge_kernel(chunks, tuple(range(8)))
    edge1 = _make_edge_kernel(chunks, (0,) * 8)
    r8 = _r_mat(8, 16)
    r1 = _r_mat(1, D)

    h0p, as_0, ad_0 = _tc_prep(x, W0, _sd_mat(as0, ad0))
    num0, den0 = edge8(h0p, as_0, ad_0, pk, zn, zd)
    h1, h1p, as_1, ad_1 = _tc_post(x, num0, den0, r8, b0, g0, be0,
                                   W1, _sd_mat(as1, ad1))
    num1, den1 = edge8(h1p, as_1, ad_1, pk, zn, zd)
    h2, h2p, as_2, ad_2 = _tc_post(h1, num1, den1, r8, b1, g1, be1,
                                   W2, _sd_mat(as2, ad2))
    num2, den2 = edge1(h2p, as_2, ad_2, pk, zn, zd)
    (h3,) = _tc_post(h2, num2, den2, r1, b2, g2, be2)
    return h3
